# baseline (speedup 1.0000x reference)
import functools

import jax
import jax.numpy as jnp
from jax import lax
from jax.experimental import pallas as pl
from jax.experimental.pallas import tpu as pltpu
from jax.experimental.pallas import tpu_sc as plsc

_NB, _NV, _NR, _ND, _NC, _NF = 1, 10000, 4, 8, 128, 64
_NW = 32
_CH = 80
_RING = 5
_LOOK = 2

_PIECES = (1600, 2400, 2400, 2400, 1200)
_VB = 400
_RJ = _NR * _ND


def _sc_gather(table, idx):
    width = table.shape[1]
    nidx = idx.shape[0]
    rpw = nidx // _NW
    nch = rpw // _CH
    ngrp = nch // _RING
    assert rpw % _CH == 0 and nch % _RING == 0

    @functools.partial(
        pl.kernel,
        mesh=plsc.VectorSubcoreMesh(core_axis_name="c", subcore_axis_name="s"),
        out_type=jax.ShapeDtypeStruct((nidx, width), table.dtype),
        scratch_types=[pltpu.VMEM((rpw,), jnp.int32)]
        + [pltpu.VMEM((_CH, width), table.dtype) for _ in range(_RING)]
        + [pltpu.SemaphoreType.DMA for _ in range(_RING)]
        + [pltpu.SemaphoreType.DMA for _ in range(_RING)],
    )
    def gather_kernel(table_hbm, idx_hbm, out_hbm, idx_v, *bufs_sems):
        rows = bufs_sems[:_RING]
        gsem = bufs_sems[_RING:2 * _RING]
        wsem = bufs_sems[2 * _RING:]
        cid = lax.axis_index("c")
        sid = lax.axis_index("s")
        wid = sid * 2 + cid
        base = wid * rpw
        pltpu.sync_copy(idx_hbm.at[pl.ds(base, rpw)], idx_v)

        def gather_copy(chunk, b):
            return pltpu.make_async_copy(
                table_hbm.at[idx_v.at[pl.ds(chunk * _CH, _CH)]], rows[b], gsem[b])

        def write_copy(chunk, b):
            return pltpu.make_async_copy(
                rows[b], out_hbm.at[pl.ds(base + chunk * _CH, _CH)], wsem[b])

        for b in range(_RING):
            gather_copy(b, b).start()

        def body(g, _):
            for b in range(_RING):
                chunk = g * _RING + b
                gather_copy(chunk, b).wait()
                wcopy = write_copy(chunk, b)
                wcopy.start()
                wcopy.wait()
                nxt = chunk + _RING

                @pl.when(nxt < nch)
                def _():
                    gather_copy(nxt, b).start()

            return ()

        lax.fori_loop(0, ngrp, body, ())

    return gather_kernel(table, idx)


def _w_build_body(k_ref, w_ref):
    for rj in range(_RJ):
        r, j = divmod(rj, _ND)
        for d in range(_ND):
            src = r * _ND + (j - d) % _ND
            w_ref[rj, :, d * _NF:(d + 1) * _NF] = k_ref[src].astype(jnp.bfloat16)


def _build_w(k2):
    return pl.pallas_call(
        _w_build_body,
        out_shape=jax.ShapeDtypeStruct((_RJ, _NC, _ND * _NF), jnp.bfloat16),
    )(k2)


def _tc_body(g_ref, y_ref, w_ref, ck_ref, b_ref, o_ref):
    acc = jnp.zeros((_VB, _ND * _NF), jnp.float32)
    for rj in range(_RJ):
        acc = acc + jnp.dot(g_ref[rj].astype(jnp.bfloat16), w_ref[rj],
                            preferred_element_type=jnp.float32,
                            precision=lax.Precision.DEFAULT)
    m = jnp.maximum(acc[:, : 4 * _NF], acc[:, 4 * _NF:])
    m = jnp.maximum(m[:, : 2 * _NF], m[:, 2 * _NF:])
    m = jnp.maximum(m[:, :_NF], m[:, _NF:])
    cent = jnp.dot(y_ref[...], ck_ref[...], preferred_element_type=jnp.float32)
    o_ref[...] = jnp.maximum(m + cent + b_ref[...], 0.0)


def _tc_conv(g3, y2, w3, ck, bias2):
    nvp = g3.shape[1]
    grid = (nvp // _VB,)
    return pl.pallas_call(
        _tc_body,
        grid=grid,
        in_specs=[
            pl.BlockSpec((_RJ, _VB, _NC), lambda i: (0, i, 0)),
            pl.BlockSpec((_VB, _NC), lambda i: (i, 0)),
            pl.BlockSpec((_RJ, _NC, _ND * _NF), lambda i: (0, 0, 0)),
            pl.BlockSpec((_NC, _NF), lambda i: (0, 0)),
            pl.BlockSpec((1, _NF), lambda i: (0, 0)),
        ],
        out_specs=pl.BlockSpec((_VB, _NF), lambda i: (i, 0)),
        out_shape=jax.ShapeDtypeStruct((nvp, _NF), jnp.float32),
    )(g3, y2, w3, ck, bias2)


def kernel(y, exp_map, kernel, center_kernel, bias):
    nb, nv, nc = y.shape
    nr, nd, _, nf = kernel.shape
    y2 = y.reshape(nb * nv, nc)
    idx3 = jnp.transpose(
        exp_map[..., 0] * nv + exp_map[..., 1], (0, 2, 3, 1)
    ).reshape(nr * nd, nb * nv).astype(jnp.int32)
    w3 = _build_w(kernel.reshape(nr * nd, nc, nf))
    bias2 = bias.reshape(1, nf)
    outs = []
    v0 = 0
    for pz in _PIECES:
        sl = slice(v0, v0 + pz)
        g3 = _sc_gather(y2, idx3[:, sl].reshape(-1)).reshape(nr * nd, pz, nc)
        outs.append(_tc_conv(g3, y2[sl], w3, center_kernel, bias2))
        v0 += pz
    out = jnp.concatenate(outs, axis=0)
    return out.reshape(nb, nv, nf)


# speedup vs baseline: 1.0175x; 1.0175x over previous
import functools

import jax
import jax.numpy as jnp
from jax import lax
from jax.experimental import pallas as pl
from jax.experimental.pallas import tpu as pltpu
from jax.experimental.pallas import tpu_sc as plsc

_NB, _NV, _NR, _ND, _NC, _NF = 1, 10000, 4, 8, 128, 64
_NW = 32
_CH = 80
_RING = 5

_PIECES = (2000, 2000, 2000, 2000, 2000)
_VB = 400
_RJ = _NR * _ND


def _sc_gather(table, idx):
    width = table.shape[1]
    nidx = idx.shape[0]
    rpw = nidx // _NW
    nch = rpw // _CH
    ngrp = nch // _RING
    assert rpw % _CH == 0 and nch % _RING == 0

    @functools.partial(
        pl.kernel,
        mesh=plsc.VectorSubcoreMesh(core_axis_name="c", subcore_axis_name="s"),
        out_type=jax.ShapeDtypeStruct((nidx, width), table.dtype),
        scratch_types=[pltpu.VMEM((rpw,), jnp.int32)]
        + [pltpu.VMEM((_CH, width), table.dtype) for _ in range(_RING)]
        + [pltpu.SemaphoreType.DMA for _ in range(_RING)]
        + [pltpu.SemaphoreType.DMA for _ in range(_RING)],
    )
    def gather_kernel(table_hbm, idx_hbm, out_hbm, idx_v, *bufs_sems):
        rows = bufs_sems[:_RING]
        gsem = bufs_sems[_RING:2 * _RING]
        wsem = bufs_sems[2 * _RING:]
        cid = lax.axis_index("c")
        sid = lax.axis_index("s")
        wid = sid * 2 + cid
        base = wid * rpw
        pltpu.sync_copy(idx_hbm.at[pl.ds(base, rpw)], idx_v)

        def gather_copy(chunk, b):
            return pltpu.make_async_copy(
                table_hbm.at[idx_v.at[pl.ds(chunk * _CH, _CH)]], rows[b], gsem[b])

        def write_copy(chunk, b):
            return pltpu.make_async_copy(
                rows[b], out_hbm.at[pl.ds(base + chunk * _CH, _CH)], wsem[b])

        for b in range(_RING):
            gather_copy(b, b).start()

        def body(g, _):
            for b in range(_RING):
                chunk = g * _RING + b
                gather_copy(chunk, b).wait()
                wcopy = write_copy(chunk, b)
                wcopy.start()
                wcopy.wait()
                nxt = chunk + _RING

                @pl.when(nxt < nch)
                def _():
                    gather_copy(nxt, b).start()

            return ()

        lax.fori_loop(0, ngrp, body, ())

    return gather_kernel(table, idx)


def _w_build_body(k_ref, w_ref):
    for rj in range(_RJ):
        r, j = divmod(rj, _ND)
        for d in range(_ND):
            src = r * _ND + (j - d) % _ND
            w_ref[rj, :, d * _NF:(d + 1) * _NF] = k_ref[src].astype(jnp.bfloat16)


def _build_w(k2):
    return pl.pallas_call(
        _w_build_body,
        out_shape=jax.ShapeDtypeStruct((_RJ, _NC, _ND * _NF), jnp.bfloat16),
    )(k2)


def _tc_body(g_ref, y_ref, w_ref, ck_ref, b_ref, o_ref):
    acc = jnp.zeros((_VB, _ND * _NF), jnp.float32)
    for rj in range(_RJ):
        acc = acc + jnp.dot(g_ref[rj].astype(jnp.bfloat16), w_ref[rj],
                            preferred_element_type=jnp.float32,
                            precision=lax.Precision.DEFAULT)
    m = jnp.maximum(acc[:, : 4 * _NF], acc[:, 4 * _NF:])
    m = jnp.maximum(m[:, : 2 * _NF], m[:, 2 * _NF:])
    m = jnp.maximum(m[:, :_NF], m[:, _NF:])
    cent = jnp.dot(y_ref[...], ck_ref[...], preferred_element_type=jnp.float32)
    o_ref[...] = jnp.maximum(m + cent + b_ref[...], 0.0)


def _tc_conv(g3, y2, w3, ck, bias2):
    nvp = g3.shape[1]
    grid = (nvp // _VB,)
    return pl.pallas_call(
        _tc_body,
        grid=grid,
        in_specs=[
            pl.BlockSpec((_RJ, _VB, _NC), lambda i: (0, i, 0)),
            pl.BlockSpec((_VB, _NC), lambda i: (i, 0)),
            pl.BlockSpec((_RJ, _NC, _ND * _NF), lambda i: (0, 0, 0)),
            pl.BlockSpec((_NC, _NF), lambda i: (0, 0)),
            pl.BlockSpec((1, _NF), lambda i: (0, 0)),
        ],
        out_specs=pl.BlockSpec((_VB, _NF), lambda i: (i, 0)),
        out_shape=jax.ShapeDtypeStruct((nvp, _NF), jnp.float32),
    )(g3, y2, w3, ck, bias2)


def kernel(y, exp_map, kernel, center_kernel, bias):
    nb, nv, nc = y.shape
    nr, nd, _, nf = kernel.shape
    y2 = y.reshape(nb * nv, nc)
    idx3 = jnp.transpose(
        exp_map[..., 0] * nv + exp_map[..., 1], (0, 2, 3, 1)
    ).reshape(nr * nd, nb * nv).astype(jnp.int32)
    w3 = _build_w(kernel.reshape(nr * nd, nc, nf))
    bias2 = bias.reshape(1, nf)
    outs = []
    v0 = 0
    for pz in _PIECES:
        sl = slice(v0, v0 + pz)
        g3 = _sc_gather(y2, idx3[:, sl].reshape(-1)).reshape(nr * nd, pz, nc)
        outs.append(_tc_conv(g3, y2[sl], w3, center_kernel, bias2))
        v0 += pz
    out = jnp.concatenate(outs, axis=0)
    return out.reshape(nb, nv, nf)
